# baseline (speedup 1.0000x reference)
import functools

import jax
import jax.numpy as jnp
from jax import lax
from jax.experimental import pallas as pl
from jax.experimental.pallas import tpu as pltpu
from jax.experimental.pallas import tpu_sc as plsc

N = 10000
E = 320000
IN_CH = 128
HID_CH = 128
OUT_CH = 64

NC = 2
NS = 16
NW = NC * NS
CHUNK = 128
CHUNKS = 80
GRP = 8
NGRP = CHUNKS // GRP
EPW = CHUNK * CHUNKS
E_PAD = EPW * NW
ROWS_PER_TILE = 632
N_TAB = ROWS_PER_TILE * NS
TRASH = N
DEG_W = 16

_MESH = plsc.VectorSubcoreMesh(core_axis_name="c", subcore_axis_name="s")

_ZB = 128


def _fill_buf(buf, value):
    d = buf.shape[1]
    v16 = jnp.full((16,), value, jnp.float32)
    for r in range(_ZB):
        for col in range(d // 16):
            buf[r, pl.ds(col * 16, 16)] = v16


def _zero_table_slice(zbuf, table, row0):
    _fill_buf(zbuf, 0.0)
    nfull, rem = ROWS_PER_TILE // _ZB, ROWS_PER_TILE % _ZB
    for b in range(nfull):
        pltpu.sync_copy(zbuf, table.at[pl.ds(row0 + b * _ZB, _ZB)])
    if rem:
        pltpu.sync_copy(zbuf.at[pl.ds(0, rem)],
                        table.at[pl.ds(row0 + nfull * _ZB, rem)])


@functools.partial(
    pl.kernel,
    out_type=jax.ShapeDtypeStruct((NC, N_TAB, DEG_W), jnp.float32),
    mesh=_MESH,
    scratch_types=[
        pltpu.VMEM((GRP, CHUNK), jnp.int32),
        pltpu.VMEM((CHUNK, DEG_W), jnp.float32),
        pltpu.VMEM((_ZB, DEG_W), jnp.float32),
        pltpu.VMEM_SHARED((N_TAB, DEG_W), jnp.float32),
        pltpu.SemaphoreType.DMA,
    ],
    compiler_params=pltpu.CompilerParams(use_tc_tiling_on_sc=False),
)
def _deg_kernel(dst_hbm, out_hbm, dst_v, ones_v, zbuf, table, sem):
    c = lax.axis_index("c")
    s = lax.axis_index("s")
    w = s * NC + c
    _fill_buf(ones_v, 1.0)
    row0 = s * ROWS_PER_TILE
    _zero_table_slice(zbuf, table, row0)
    plsc.subcore_barrier()

    def group(g, carry):
        pltpu.sync_copy(dst_hbm.at[w, pl.ds(g * GRP, GRP)], dst_v)
        for j in range(GRP):
            pltpu.async_copy(ones_v, table.at[dst_v.at[j]], sem, add=True)
        for j in range(GRP):
            pltpu.make_async_copy(ones_v, table.at[dst_v.at[j]], sem).wait()
        return carry

    lax.fori_loop(0, NGRP, group, 0)
    plsc.subcore_barrier()
    pltpu.sync_copy(
        table.at[pl.ds(row0, ROWS_PER_TILE)],
        out_hbm.at[c, pl.ds(row0, ROWS_PER_TILE)],
    )


def _make_agg_kernel(d, tc_tiling=True, stage_u=False):
    scratch = [
        pltpu.VMEM((GRP, CHUNK), jnp.int32),
        pltpu.VMEM((GRP, CHUNK), jnp.int32),
        pltpu.VMEM((CHUNK, d), jnp.float32),
        pltpu.VMEM((CHUNK, d), jnp.float32),
        pltpu.VMEM_SHARED((N_TAB, d), jnp.float32),
        pltpu.SemaphoreType.DMA,
        pltpu.SemaphoreType.DMA,
        pltpu.SemaphoreType.DMA,
        pltpu.SemaphoreType.DMA,
    ]
    if stage_u:
        scratch.append(pltpu.VMEM_SHARED((N_TAB, d), jnp.float32))

    @functools.partial(
        pl.kernel,
        out_type=jax.ShapeDtypeStruct((NC, N_TAB, d), jnp.float32),
        mesh=_MESH,
        compiler_params=pltpu.CompilerParams(use_tc_tiling_on_sc=tc_tiling),
        scratch_types=scratch,
    )
    def _agg(u_hbm_in, src_hbm, dst_hbm, out_hbm,
             src_v, dst_v, buf0, buf1, table, sem0, sem1, semc0, semc1,
             *maybe_uspm):
        c = lax.axis_index("c")
        s = lax.axis_index("s")
        w = s * NC + c
        row0 = s * ROWS_PER_TILE
        _zero_table_slice(buf0, table, row0)
        if stage_u:
            u_hbm = maybe_uspm[0]
            ntail = N - (NS - 1) * ROWS_PER_TILE

            @pl.when(s < NS - 1)
            def _():
                pltpu.sync_copy(u_hbm_in.at[pl.ds(row0, ROWS_PER_TILE)],
                                u_hbm.at[pl.ds(row0, ROWS_PER_TILE)])

            @pl.when(s == NS - 1)
            def _():
                pltpu.sync_copy(u_hbm_in.at[pl.ds(row0, ntail)],
                                u_hbm.at[pl.ds(row0, ntail)])
        else:
            u_hbm = u_hbm_in
        plsc.subcore_barrier()

        bufs = (buf0, buf1)
        sems = (sem0, sem1)
        semcs = (semc0, semc1)

        def wait_scatter(j):
            pltpu.make_async_copy(bufs[j % 2], table.at[dst_v.at[j]],
                                  semcs[j % 2]).wait()

        def group(g, carry):
            pltpu.sync_copy(src_hbm.at[w, pl.ds(g * GRP, GRP)], src_v)
            pltpu.sync_copy(dst_hbm.at[w, pl.ds(g * GRP, GRP)], dst_v)
            pltpu.async_copy(u_hbm.at[src_v.at[0]], buf0, sem0)
            for j in range(GRP):
                if j + 1 < GRP:
                    if j >= 1:
                        wait_scatter(j - 1)
                    pltpu.async_copy(u_hbm.at[src_v.at[j + 1]],
                                     bufs[(j + 1) % 2], sems[(j + 1) % 2])
                pltpu.make_async_copy(u_hbm.at[src_v.at[j]],
                                      bufs[j % 2], sems[j % 2]).wait()
                pltpu.async_copy(bufs[j % 2], table.at[dst_v.at[j]],
                                 semcs[j % 2], add=True)
            wait_scatter(GRP - 2)
            wait_scatter(GRP - 1)
            return carry

        lax.fori_loop(0, NGRP, group, 0)

        plsc.subcore_barrier()
        pltpu.sync_copy(
            table.at[pl.ds(row0, ROWS_PER_TILE)],
            out_hbm.at[c, pl.ds(row0, ROWS_PER_TILE)],
        )

    return _agg


_agg128 = _make_agg_kernel(HID_CH)
_agg64 = _make_agg_kernel(OUT_CH, tc_tiling=False, stage_u=True)


_R = 2000


def _dinv_block(dega_ref, degb_ref):
    deg = dega_ref[:, 0:1] + degb_ref[:, 0:1] + 1.0
    return lax.rsqrt(deg)


def _mm1_body(x_ref, w_ref, h_ref):
    h_ref[...] = jnp.dot(x_ref[...], w_ref[...],
                         preferred_element_type=jnp.float32)


def _scale_body(h_ref, dega_ref, degb_ref, u_ref):
    u_ref[...] = h_ref[...] * _dinv_block(dega_ref, degb_ref)


def _mid_body(agg0_ref, agg1_ref, u1_ref, dega_ref, degb_ref, b1_ref, w2_ref,
              u2_ref):
    dinv = _dinv_block(dega_ref, degb_ref)
    t = (agg0_ref[...] + agg1_ref[...] + u1_ref[...]) * dinv + b1_ref[...]
    h1 = jnp.maximum(t, 0.0)
    u2_ref[...] = jnp.dot(h1, w2_ref[...],
                          preferred_element_type=jnp.float32) * dinv


def _fin_body(agg0_ref, agg1_ref, u2_ref, dega_ref, degb_ref, b2_ref, o_ref):
    dinv = _dinv_block(dega_ref, degb_ref)
    z = (agg0_ref[...] + agg1_ref[...] + u2_ref[...]) * dinv + b2_ref[...]
    m = jnp.max(z, axis=1, keepdims=True)
    lse = jnp.log(jnp.sum(jnp.exp(z - m), axis=1, keepdims=True)) + m
    o_ref[...] = z - lse


def _row_spec(width):
    return pl.BlockSpec((_R, width), lambda i: (i, 0))


def _full_spec(shape):
    return pl.BlockSpec(shape, lambda i: tuple(0 for _ in shape))


def _mm1(x, w1):
    return pl.pallas_call(
        _mm1_body,
        grid=(N // _R,),
        in_specs=[_row_spec(IN_CH), _full_spec((IN_CH, HID_CH))],
        out_specs=_row_spec(HID_CH),
        out_shape=jax.ShapeDtypeStruct((N, HID_CH), jnp.float32),
    )(x, w1)


def _scale(h, dega, degb):
    return pl.pallas_call(
        _scale_body,
        grid=(N // _R,),
        in_specs=[
            _row_spec(HID_CH),
            _row_spec(DEG_W),
            _row_spec(DEG_W),
        ],
        out_specs=_row_spec(HID_CH),
        out_shape=jax.ShapeDtypeStruct((N, HID_CH), jnp.float32),
    )(h, dega, degb)


def _mid(agg0, agg1, u1, dega, degb, b1, w2):
    return pl.pallas_call(
        _mid_body,
        grid=(N // _R,),
        in_specs=[
            _row_spec(HID_CH),
            _row_spec(HID_CH),
            _row_spec(HID_CH),
            _row_spec(DEG_W),
            _row_spec(DEG_W),
            _full_spec((1, HID_CH)),
            _full_spec((HID_CH, OUT_CH)),
        ],
        out_specs=_row_spec(OUT_CH),
        out_shape=jax.ShapeDtypeStruct((N, OUT_CH), jnp.float32),
    )(agg0, agg1, u1, dega, degb, b1, w2)


def _fin(agg0, agg1, u2, dega, degb, b2):
    return pl.pallas_call(
        _fin_body,
        grid=(N // _R,),
        in_specs=[
            _row_spec(OUT_CH),
            _row_spec(OUT_CH),
            _row_spec(OUT_CH),
            _row_spec(DEG_W),
            _row_spec(DEG_W),
            _full_spec((1, OUT_CH)),
        ],
        out_specs=_row_spec(OUT_CH),
        out_shape=jax.ShapeDtypeStruct((N, OUT_CH), jnp.float32),
    )(agg0, agg1, u2, dega, degb, b2)


@jax.jit
def kernel(x, edge_index, W1, b1, W2, b2):
    src = edge_index[0].astype(jnp.int32)
    dst = edge_index[1].astype(jnp.int32)
    pad = E_PAD - E
    pad_i = jnp.arange(pad, dtype=jnp.int32)
    src_p = jnp.concatenate(
        [src, pad_i % N]).reshape(NW, CHUNKS, CHUNK)
    dst_p = jnp.concatenate(
        [dst, TRASH + pad_i % (N_TAB - N)]).reshape(NW, CHUNKS, CHUNK)

    h1raw = _mm1(x, W1)
    deg_parts = _deg_kernel(dst_p)
    dega, degb = deg_parts[0], deg_parts[1]

    u1 = _scale(h1raw, dega, degb)
    agg1 = _agg128(u1, src_p, dst_p)
    u2 = _mid(agg1[0, :N], agg1[1, :N], u1, dega, degb,
              b1.reshape(1, HID_CH), W2)
    agg2 = _agg64(u2, src_p, dst_p)
    return _fin(agg2[0, :N], agg2[1, :N], u2,
                dega, degb, b2.reshape(1, OUT_CH))


# speedup vs baseline: 1.0558x; 1.0558x over previous
import functools

import jax
import jax.numpy as jnp
from jax import lax
from jax.experimental import pallas as pl
from jax.experimental.pallas import tpu as pltpu
from jax.experimental.pallas import tpu_sc as plsc

N = 10000
E = 320000
IN_CH = 128
HID_CH = 128
OUT_CH = 64

NC = 2
NS = 16
NW = NC * NS
CHUNK = 128
CHUNKS = 80
GRP = 8
NGRP = CHUNKS // GRP
EPW = CHUNK * CHUNKS
E_PAD = EPW * NW
ROWS_PER_TILE = 632
N_TAB = ROWS_PER_TILE * NS
TRASH = N
DEG_W = 16

_MESH = plsc.VectorSubcoreMesh(core_axis_name="c", subcore_axis_name="s")

_ZB = 128


def _fill_buf(buf, value):
    d = buf.shape[1]
    v16 = jnp.full((16,), value, jnp.float32)
    for r in range(_ZB):
        for col in range(d // 16):
            buf[r, pl.ds(col * 16, 16)] = v16


def _zero_table_slice(zbuf, table, row0):
    _fill_buf(zbuf, 0.0)
    nfull, rem = ROWS_PER_TILE // _ZB, ROWS_PER_TILE % _ZB
    for b in range(nfull):
        pltpu.sync_copy(zbuf, table.at[pl.ds(row0 + b * _ZB, _ZB)])
    if rem:
        pltpu.sync_copy(zbuf.at[pl.ds(0, rem)],
                        table.at[pl.ds(row0 + nfull * _ZB, rem)])


@functools.partial(
    pl.kernel,
    out_type=jax.ShapeDtypeStruct((NC, N_TAB, DEG_W), jnp.float32),
    mesh=_MESH,
    scratch_types=[
        pltpu.VMEM((GRP, CHUNK), jnp.int32),
        pltpu.VMEM((CHUNK, DEG_W), jnp.float32),
        pltpu.VMEM((_ZB, DEG_W), jnp.float32),
        pltpu.VMEM_SHARED((N_TAB, DEG_W), jnp.float32),
        pltpu.SemaphoreType.DMA,
    ],
    compiler_params=pltpu.CompilerParams(use_tc_tiling_on_sc=False),
)
def _deg_kernel(dst_hbm, out_hbm, dst_v, ones_v, zbuf, table, sem):
    c = lax.axis_index("c")
    s = lax.axis_index("s")
    w = s * NC + c
    _fill_buf(ones_v, 1.0)
    row0 = s * ROWS_PER_TILE
    _zero_table_slice(zbuf, table, row0)
    plsc.subcore_barrier()

    def group(g, carry):
        pltpu.sync_copy(dst_hbm.at[w, pl.ds(g * GRP, GRP)], dst_v)
        for j in range(GRP):
            pltpu.async_copy(ones_v, table.at[dst_v.at[j]], sem, add=True)
        for j in range(GRP):
            pltpu.make_async_copy(ones_v, table.at[dst_v.at[j]], sem).wait()
        return carry

    lax.fori_loop(0, NGRP, group, 0)
    plsc.subcore_barrier()
    pltpu.sync_copy(
        table.at[pl.ds(row0, ROWS_PER_TILE)],
        out_hbm.at[c, pl.ds(row0, ROWS_PER_TILE)],
    )


def _make_agg_kernel(d, tc_tiling=True):
    @functools.partial(
        pl.kernel,
        out_type=jax.ShapeDtypeStruct((NC, N_TAB, d), jnp.float32),
        mesh=_MESH,
        compiler_params=pltpu.CompilerParams(use_tc_tiling_on_sc=tc_tiling),
        scratch_types=[
            pltpu.VMEM((GRP, CHUNK), jnp.int32),
            pltpu.VMEM((GRP, CHUNK), jnp.int32),
            pltpu.VMEM((CHUNK, d), jnp.float32),
            pltpu.VMEM((CHUNK, d), jnp.float32),
            pltpu.VMEM_SHARED((N_TAB, d), jnp.float32),
            pltpu.SemaphoreType.DMA,
            pltpu.SemaphoreType.DMA,
            pltpu.SemaphoreType.DMA,
            pltpu.SemaphoreType.DMA,
        ],
    )
    def _agg(u_hbm, src_hbm, dst_hbm, out_hbm,
             src_v, dst_v, buf0, buf1, table, sem0, sem1, semc0, semc1):
        c = lax.axis_index("c")
        s = lax.axis_index("s")
        w = s * NC + c
        row0 = s * ROWS_PER_TILE
        _zero_table_slice(buf0, table, row0)
        plsc.subcore_barrier()

        bufs = (buf0, buf1)
        sems = (sem0, sem1)
        semcs = (semc0, semc1)

        def wait_scatter(j):
            pltpu.make_async_copy(bufs[j % 2], table.at[dst_v.at[j]],
                                  semcs[j % 2]).wait()

        def group(g, carry):
            pltpu.sync_copy(src_hbm.at[w, pl.ds(g * GRP, GRP)], src_v)
            pltpu.sync_copy(dst_hbm.at[w, pl.ds(g * GRP, GRP)], dst_v)
            pltpu.async_copy(u_hbm.at[src_v.at[0]], buf0, sem0)
            for j in range(GRP):
                if j + 1 < GRP:
                    if j >= 1:
                        wait_scatter(j - 1)
                    pltpu.async_copy(u_hbm.at[src_v.at[j + 1]],
                                     bufs[(j + 1) % 2], sems[(j + 1) % 2])
                pltpu.make_async_copy(u_hbm.at[src_v.at[j]],
                                      bufs[j % 2], sems[j % 2]).wait()
                pltpu.async_copy(bufs[j % 2], table.at[dst_v.at[j]],
                                 semcs[j % 2], add=True)
            wait_scatter(GRP - 2)
            wait_scatter(GRP - 1)
            return carry

        lax.fori_loop(0, NGRP, group, 0)

        plsc.subcore_barrier()
        pltpu.sync_copy(
            table.at[pl.ds(row0, ROWS_PER_TILE)],
            out_hbm.at[c, pl.ds(row0, ROWS_PER_TILE)],
        )

    return _agg


_agg128 = _make_agg_kernel(HID_CH)
_agg64 = _make_agg_kernel(OUT_CH, tc_tiling=False)


_R = 2000


def _dinv_block(dega_ref, degb_ref):
    deg = dega_ref[:, 0:1] + degb_ref[:, 0:1] + 1.0
    return lax.rsqrt(deg)


def _mm1_body(x_ref, w_ref, h_ref):
    h_ref[...] = jnp.dot(x_ref[...], w_ref[...],
                         preferred_element_type=jnp.float32)


def _scale_body(h_ref, dega_ref, degb_ref, u_ref):
    u_ref[...] = h_ref[...] * _dinv_block(dega_ref, degb_ref)


def _mid_body(agg0_ref, agg1_ref, u1_ref, dega_ref, degb_ref, b1_ref, w2_ref,
              u2_ref):
    dinv = _dinv_block(dega_ref, degb_ref)
    t = (agg0_ref[...] + agg1_ref[...] + u1_ref[...]) * dinv + b1_ref[...]
    h1 = jnp.maximum(t, 0.0)
    u2_ref[...] = jnp.dot(h1, w2_ref[...],
                          preferred_element_type=jnp.float32) * dinv


def _fin_body(agg0_ref, agg1_ref, u2_ref, dega_ref, degb_ref, b2_ref, o_ref):
    dinv = _dinv_block(dega_ref, degb_ref)
    z = (agg0_ref[...] + agg1_ref[...] + u2_ref[...]) * dinv + b2_ref[...]
    m = jnp.max(z, axis=1, keepdims=True)
    lse = jnp.log(jnp.sum(jnp.exp(z - m), axis=1, keepdims=True)) + m
    o_ref[...] = z - lse


def _row_spec(width):
    return pl.BlockSpec((_R, width), lambda i: (i, 0))


def _full_spec(shape):
    return pl.BlockSpec(shape, lambda i: tuple(0 for _ in shape))


def _mm1(x, w1):
    return pl.pallas_call(
        _mm1_body,
        grid=(N // _R,),
        in_specs=[_row_spec(IN_CH), _full_spec((IN_CH, HID_CH))],
        out_specs=_row_spec(HID_CH),
        out_shape=jax.ShapeDtypeStruct((N, HID_CH), jnp.float32),
    )(x, w1)


def _scale(h, dega, degb):
    return pl.pallas_call(
        _scale_body,
        grid=(N // _R,),
        in_specs=[
            _row_spec(HID_CH),
            _row_spec(DEG_W),
            _row_spec(DEG_W),
        ],
        out_specs=_row_spec(HID_CH),
        out_shape=jax.ShapeDtypeStruct((N, HID_CH), jnp.float32),
    )(h, dega, degb)


def _mid(agg0, agg1, u1, dega, degb, b1, w2):
    return pl.pallas_call(
        _mid_body,
        grid=(N // _R,),
        in_specs=[
            _row_spec(HID_CH),
            _row_spec(HID_CH),
            _row_spec(HID_CH),
            _row_spec(DEG_W),
            _row_spec(DEG_W),
            _full_spec((1, HID_CH)),
            _full_spec((HID_CH, OUT_CH)),
        ],
        out_specs=_row_spec(OUT_CH),
        out_shape=jax.ShapeDtypeStruct((N, OUT_CH), jnp.float32),
    )(agg0, agg1, u1, dega, degb, b1, w2)


def _fin(agg0, agg1, u2, dega, degb, b2):
    return pl.pallas_call(
        _fin_body,
        grid=(N // _R,),
        in_specs=[
            _row_spec(OUT_CH),
            _row_spec(OUT_CH),
            _row_spec(OUT_CH),
            _row_spec(DEG_W),
            _row_spec(DEG_W),
            _full_spec((1, OUT_CH)),
        ],
        out_specs=_row_spec(OUT_CH),
        out_shape=jax.ShapeDtypeStruct((N, OUT_CH), jnp.float32),
    )(agg0, agg1, u2, dega, degb, b2)


@jax.jit
def kernel(x, edge_index, W1, b1, W2, b2):
    src = edge_index[0].astype(jnp.int32)
    dst = edge_index[1].astype(jnp.int32)
    pad = E_PAD - E
    pad_i = jnp.arange(pad, dtype=jnp.int32)
    src_p = jnp.concatenate(
        [src, pad_i % N]).reshape(NW, CHUNKS, CHUNK)
    dst_p = jnp.concatenate(
        [dst, TRASH + pad_i % (N_TAB - N)]).reshape(NW, CHUNKS, CHUNK)

    h1raw = _mm1(x, W1)
    deg_parts = _deg_kernel(dst_p)
    dega, degb = deg_parts[0], deg_parts[1]

    u1 = _scale(h1raw, dega, degb)
    agg1 = _agg128(u1, src_p, dst_p)
    u2 = _mid(agg1[0, :N], agg1[1, :N], u1, dega, degb,
              b1.reshape(1, HID_CH), W2)
    agg2 = _agg64(u2, src_p, dst_p)
    return _fin(agg2[0, :N], agg2[1, :N], u2,
                dega, degb, b2.reshape(1, OUT_CH))
